# Initial kernel scaffold; baseline (speedup 1.0000x reference)
import jax
import jax.numpy as jnp
from jax.experimental import pallas as pl


def kernel(x, in_degree, out_degree, z_in, z_out):
    raise NotImplementedError("write your pallas kernel here")


# speedup vs baseline: 1.7527x; 1.7527x over previous
import functools

import jax
import jax.numpy as jnp
from jax import lax
from jax.experimental import pallas as pl
from jax.experimental.pallas import tpu as pltpu
from jax.experimental.pallas import tpu_sc as plsc

N = 100000
D = 128
B = 40
C = N // B
NC = 2
NS = 16
NW = NC * NS
LANES = 16

_mesh = plsc.VectorSubcoreMesh(core_axis_name="c", subcore_axis_name="s")


@functools.partial(
    pl.kernel,
    out_type=jax.ShapeDtypeStruct((N, D), jnp.float32),
    mesh=_mesh,
    scratch_types=[
        pltpu.VMEM((B,), jnp.int32),
        pltpu.VMEM((B,), jnp.int32),
        pltpu.VMEM((B, D), jnp.float32),
        pltpu.VMEM((B, D), jnp.float32),
        pltpu.VMEM((B, D), jnp.float32),
        pltpu.SemaphoreType.DMA,
        pltpu.SemaphoreType.DMA,
    ],
)
def _sc_kernel(x_hbm, ind_hbm, outd_hbm, zin_hbm, zout_hbm, o_hbm,
               idx_i, idx_o, xb, zi, zo, sem_i, sem_o):
    wid = lax.axis_index("s") * NC + lax.axis_index("c")
    n_chunks = (C - 1 - wid) // NW + 1

    def chunk_body(k, _):
        base = (wid + k * NW) * B
        pltpu.sync_copy(ind_hbm.at[pl.ds(base, B)], idx_i)
        pltpu.sync_copy(outd_hbm.at[pl.ds(base, B)], idx_o)
        cp_i = pltpu.async_copy(zin_hbm.at[idx_i], zi, sem_i)
        cp_o = pltpu.async_copy(zout_hbm.at[idx_o], zo, sem_o)
        pltpu.sync_copy(x_hbm.at[pl.ds(base, B)], xb)
        cp_i.wait()
        cp_o.wait()

        def row_body(i, _2):
            for j in range(D // LANES):
                s = pl.ds(j * LANES, LANES)
                xb[i, s] = xb[i, s] + zi[i, s] + zo[i, s]
            return 0

        lax.fori_loop(0, B, row_body, 0)
        pltpu.sync_copy(xb, o_hbm.at[pl.ds(base, B)])
        return 0

    lax.fori_loop(0, n_chunks, chunk_body, 0)


def kernel(x, in_degree, out_degree, z_in, z_out):
    return _sc_kernel(x, in_degree.astype(jnp.int32),
                      out_degree.astype(jnp.int32), z_in, z_out)


# speedup vs baseline: 3.3233x; 1.8961x over previous
import functools

import jax
import jax.numpy as jnp
from jax import lax
from jax.experimental import pallas as pl
from jax.experimental.pallas import tpu as pltpu
from jax.experimental.pallas import tpu_sc as plsc

N = 100000
D = 128
B = 200
C = N // B
NC = 2
NS = 16
NW = NC * NS
V = 512
LANES = 16

_mesh = plsc.VectorSubcoreMesh(core_axis_name="c", subcore_axis_name="s")


@functools.partial(
    pl.kernel,
    out_type=jax.ShapeDtypeStruct((N, D), jnp.float32),
    mesh=_mesh,
    scratch_types=[
        pltpu.VMEM((B,), jnp.int32),
        pltpu.VMEM((B,), jnp.int32),
        pltpu.VMEM((B, D), jnp.float32),
        pltpu.VMEM((B, D), jnp.float32),
        pltpu.VMEM((B, D), jnp.float32),
        pltpu.VMEM_SHARED((V, D), jnp.float32),
        pltpu.VMEM_SHARED((V, D), jnp.float32),
        pltpu.SemaphoreType.DMA,
        pltpu.SemaphoreType.DMA,
    ],
)
def _sc_kernel(x_hbm, ind_hbm, outd_hbm, zin_hbm, zout_hbm, o_hbm,
               idx_i, idx_o, xb, zi, zo, sh_zin, sh_zout, sem_i, sem_o):
    cid = lax.axis_index("c")
    sid = lax.axis_index("s")
    wid = sid * NC + cid

    @pl.when(sid == 0)
    def _stage():
        pltpu.sync_copy(zin_hbm, sh_zin)
        pltpu.sync_copy(zout_hbm, sh_zout)

    plsc.subcore_barrier()
    n_chunks = (C - 1 - wid) // NW + 1

    def chunk_body(k, _):
        c = wid + k * NW
        base = c * B
        pltpu.sync_copy(ind_hbm.at[pl.ds(base, B)], idx_i)
        pltpu.sync_copy(outd_hbm.at[pl.ds(base, B)], idx_o)
        cp_i = pltpu.async_copy(sh_zin.at[idx_i], zi, sem_i)
        cp_o = pltpu.async_copy(sh_zout.at[idx_o], zo, sem_o)
        pltpu.sync_copy(x_hbm.at[pl.ds(base, B)], xb)
        cp_i.wait()
        cp_o.wait()

        def row_body(i, _2):
            for j in range(D // LANES):
                s = pl.ds(j * LANES, LANES)
                xb[i, s] = xb[i, s] + zi[i, s] + zo[i, s]
            return 0

        lax.fori_loop(0, B, row_body, 0)
        pltpu.sync_copy(xb, o_hbm.at[pl.ds(base, B)])
        return 0

    lax.fori_loop(0, n_chunks, chunk_body, 0)


def kernel(x, in_degree, out_degree, z_in, z_out):
    return _sc_kernel(x, in_degree.astype(jnp.int32),
                      out_degree.astype(jnp.int32), z_in, z_out)


# speedup vs baseline: 4.5545x; 1.3705x over previous
import functools

import jax
import jax.numpy as jnp
from jax import lax
from jax.experimental import pallas as pl
from jax.experimental.pallas import tpu as pltpu
from jax.experimental.pallas import tpu_sc as plsc

N = 100000
D = 128
B = 128
C = N // B
TAIL = N - C * B
NC = 2
NS = 16
NW = NC * NS
BIG = C - (C // NW) * NW
CPW_BIG = C // NW + 1
CPW_SMALL = C // NW
PAIRS = (CPW_BIG + 1) // 2
V = 512
LANES = 16

_mesh = plsc.VectorSubcoreMesh(core_axis_name="c", subcore_axis_name="s")


@functools.partial(
    pl.kernel,
    out_type=jax.ShapeDtypeStruct((N, D), jnp.float32),
    mesh=_mesh,
    scratch_types=[
        pltpu.VMEM((CPW_BIG * B,), jnp.int32),
        pltpu.VMEM((CPW_BIG * B,), jnp.int32),
        pltpu.VMEM((TAIL,), jnp.int32),
        pltpu.VMEM((TAIL,), jnp.int32),
        pltpu.VMEM((B, D), jnp.float32),
        pltpu.VMEM((B, D), jnp.float32),
        pltpu.VMEM((B, D), jnp.float32),
        pltpu.VMEM((B, D), jnp.float32),
        pltpu.VMEM((B, D), jnp.float32),
        pltpu.VMEM((B, D), jnp.float32),
        pltpu.VMEM_SHARED((V, D), jnp.float32),
        pltpu.VMEM_SHARED((V, D), jnp.float32),
        pltpu.SemaphoreType.DMA,
        pltpu.SemaphoreType.DMA,
        pltpu.SemaphoreType.DMA,
        pltpu.SemaphoreType.DMA,
        pltpu.SemaphoreType.DMA,
        pltpu.SemaphoreType.DMA,
    ],
)
def _sc_kernel(x_hbm, ind_hbm, outd_hbm, zin_hbm, zout_hbm, o_hbm,
               idx_i, idx_o, tidx_i, tidx_o,
               xb_a, zi_a, zo_a, xb_b, zi_b, zo_b,
               sh_zin, sh_zout,
               sem_g_a, sem_x_a, sem_out_a, sem_g_b, sem_x_b, sem_out_b):
    cid = lax.axis_index("c")
    sid = lax.axis_index("s")
    wid = sid * NC + cid
    is_big = wid < BIG
    s0 = jnp.where(is_big, wid * CPW_BIG, wid * CPW_SMALL + BIG)
    n = jnp.where(is_big, CPW_BIG, CPW_SMALL)

    @pl.when(sid == 0)
    def _stage():
        pltpu.sync_copy(zin_hbm, sh_zin)
        pltpu.sync_copy(zout_hbm, sh_zout)

    @pl.when(is_big)
    def _idx_big():
        pltpu.sync_copy(ind_hbm.at[pl.ds(s0 * B, CPW_BIG * B)], idx_i)
        pltpu.sync_copy(outd_hbm.at[pl.ds(s0 * B, CPW_BIG * B)], idx_o)

    @pl.when(jnp.logical_not(is_big))
    def _idx_small():
        pltpu.sync_copy(ind_hbm.at[pl.ds(s0 * B, CPW_SMALL * B)],
                        idx_i.at[pl.ds(0, CPW_SMALL * B)])
        pltpu.sync_copy(outd_hbm.at[pl.ds(s0 * B, CPW_SMALL * B)],
                        idx_o.at[pl.ds(0, CPW_SMALL * B)])

    plsc.subcore_barrier()

    bufs = (
        dict(xb=xb_a, zi=zi_a, zo=zo_a, sem_g=sem_g_a, sem_x=sem_x_a,
             sem_out=sem_out_a),
        dict(xb=xb_b, zi=zi_b, zo=zo_b, sem_g=sem_g_b, sem_x=sem_x_b,
             sem_out=sem_out_b),
    )

    def issue_gathers(k, S):
        @pl.when(k < n)
        def _():
            pltpu.async_copy(sh_zin.at[idx_i.at[pl.ds(k * B, B)]],
                             S["zi"], S["sem_g"])
            pltpu.async_copy(sh_zout.at[idx_o.at[pl.ds(k * B, B)]],
                             S["zo"], S["sem_g"])

    def issue_x(k, S):
        @pl.when(k < n)
        def _():
            pltpu.async_copy(x_hbm.at[pl.ds((s0 + k) * B, B)],
                             S["xb"], S["sem_x"])

    def finish(k, S):
        @pl.when(k < n)
        def _():
            pltpu.make_async_copy(sh_zin.at[pl.ds(0, B)],
                                  S["zi"], S["sem_g"]).wait()
            pltpu.make_async_copy(sh_zout.at[pl.ds(0, B)],
                                  S["zo"], S["sem_g"]).wait()
            pltpu.make_async_copy(x_hbm.at[pl.ds(0, B)],
                                  S["xb"], S["sem_x"]).wait()
            xb, zi, zo = S["xb"], S["zi"], S["zo"]

            def row_body(i, _2):
                for j in range(D // LANES):
                    s = pl.ds(j * LANES, LANES)
                    zi[i, s] = xb[i, s] + zi[i, s] + zo[i, s]
                return 0

            lax.fori_loop(0, B, row_body, 0)
            pltpu.async_copy(S["zi"], o_hbm.at[pl.ds((s0 + k) * B, B)],
                             S["sem_out"])

    def drain_out(k, S):
        @pl.when(k < n)
        def _():
            pltpu.make_async_copy(S["zi"], o_hbm.at[pl.ds(0, B)],
                                  S["sem_out"]).wait()

    issue_gathers(0, bufs[0])
    issue_x(0, bufs[0])
    issue_gathers(1, bufs[1])
    issue_x(1, bufs[1])

    def pair_body(t, _):
        k0 = 2 * t
        k1 = k0 + 1
        finish(k0, bufs[0])
        issue_x(k0 + 2, bufs[0])
        finish(k1, bufs[1])
        issue_x(k1 + 2, bufs[1])
        drain_out(k0, bufs[0])
        issue_gathers(k0 + 2, bufs[0])
        drain_out(k1, bufs[1])
        issue_gathers(k1 + 2, bufs[1])
        return 0

    lax.fori_loop(0, PAIRS, pair_body, 0)

    @pl.when(wid == NW - 1)
    def _tail():
        base = C * B
        pltpu.sync_copy(ind_hbm.at[pl.ds(base, TAIL)], tidx_i)
        pltpu.sync_copy(outd_hbm.at[pl.ds(base, TAIL)], tidx_o)
        cp_i = pltpu.async_copy(sh_zin.at[tidx_i],
                                zi_a.at[pl.ds(0, TAIL)], sem_g_a)
        cp_o = pltpu.async_copy(sh_zout.at[tidx_o],
                                zo_a.at[pl.ds(0, TAIL)], sem_g_a)
        pltpu.sync_copy(x_hbm.at[pl.ds(base, TAIL)], xb_a.at[pl.ds(0, TAIL)])
        cp_i.wait()
        cp_o.wait()

        def row_body(i, _2):
            for j in range(D // LANES):
                s = pl.ds(j * LANES, LANES)
                zi_a[i, s] = xb_a[i, s] + zi_a[i, s] + zo_a[i, s]
            return 0

        lax.fori_loop(0, TAIL, row_body, 0)
        pltpu.sync_copy(zi_a.at[pl.ds(0, TAIL)], o_hbm.at[pl.ds(base, TAIL)])


def kernel(x, in_degree, out_degree, z_in, z_out):
    return _sc_kernel(x, in_degree.astype(jnp.int32),
                      out_degree.astype(jnp.int32), z_in, z_out)


# speedup vs baseline: 5.8553x; 1.2856x over previous
import functools

import jax
import jax.numpy as jnp
from jax import lax
from jax.experimental import pallas as pl
from jax.experimental.pallas import tpu as pltpu
from jax.experimental.pallas import tpu_sc as plsc

N = 100000
D = 128
B = 128
C = N // B
TAIL = N - C * B
NC = 2
NS = 16
NW = NC * NS
BIG = C - (C // NW) * NW
CPW_BIG = C // NW + 1
CPW_SMALL = C // NW
NBUF = 6
BLOCKS = (CPW_BIG + 4 + NBUF - 1) // NBUF
V = 512
LANES = 16

_mesh = plsc.VectorSubcoreMesh(core_axis_name="c", subcore_axis_name="s")


@functools.partial(
    pl.kernel,
    out_type=jax.ShapeDtypeStruct((N, D), jnp.float32),
    mesh=_mesh,
    scratch_types=[
        pltpu.VMEM((CPW_BIG * B,), jnp.int32),
        pltpu.VMEM((CPW_BIG * B,), jnp.int32),
        pltpu.VMEM((CPW_BIG * B,), jnp.int32),
        pltpu.VMEM((TAIL,), jnp.int32),
        pltpu.VMEM((TAIL,), jnp.int32),
        pltpu.VMEM((TAIL,), jnp.int32),
        [pltpu.VMEM((B, D), jnp.float32) for _ in range(NBUF)],
        [pltpu.SemaphoreType.DMA for _ in range(NBUF)],
        pltpu.VMEM_SHARED((V, D), jnp.float32),
        pltpu.VMEM_SHARED((V, D), jnp.float32),
    ],
)
def _sc_kernel(x_hbm, ind_hbm, outd_hbm, rid_hbm, zin_hbm, zout_hbm, o_hbm,
               idx_i, idx_o, ridx, tidx_i, tidx_o, tridx,
               acc, sems, sh_zin, sh_zout):
    cid = lax.axis_index("c")
    sid = lax.axis_index("s")
    wid = sid * NC + cid
    is_big = wid < BIG
    s0 = jnp.where(is_big, wid * CPW_BIG, wid * CPW_SMALL + BIG)
    n = jnp.where(is_big, CPW_BIG, CPW_SMALL)

    @pl.when(sid == 0)
    def _stage():
        pltpu.sync_copy(zin_hbm, sh_zin)
        pltpu.sync_copy(zout_hbm, sh_zout)

    @pl.when(is_big)
    def _idx_big():
        pltpu.sync_copy(ind_hbm.at[pl.ds(s0 * B, CPW_BIG * B)], idx_i)
        pltpu.sync_copy(outd_hbm.at[pl.ds(s0 * B, CPW_BIG * B)], idx_o)
        pltpu.sync_copy(rid_hbm.at[pl.ds(s0 * B, CPW_BIG * B)], ridx)

    @pl.when(jnp.logical_not(is_big))
    def _idx_small():
        pltpu.sync_copy(ind_hbm.at[pl.ds(s0 * B, CPW_SMALL * B)],
                        idx_i.at[pl.ds(0, CPW_SMALL * B)])
        pltpu.sync_copy(outd_hbm.at[pl.ds(s0 * B, CPW_SMALL * B)],
                        idx_o.at[pl.ds(0, CPW_SMALL * B)])
        pltpu.sync_copy(rid_hbm.at[pl.ds(s0 * B, CPW_SMALL * B)],
                        ridx.at[pl.ds(0, CPW_SMALL * B)])

    plsc.subcore_barrier()

    def ok(c):
        return jnp.logical_and(c >= 0, c < n)

    def s1_zin(c, buf, sem):
        @pl.when(ok(c))
        def _():
            pltpu.async_copy(sh_zin.at[idx_i.at[pl.ds(c * B, B)]], buf, sem)

    def s2_zout(c, buf, sem):
        @pl.when(ok(c))
        def _():
            pltpu.make_async_copy(sh_zin.at[pl.ds(0, B)], buf, sem).wait()
            pltpu.async_copy(sh_zout.at[idx_o.at[pl.ds(c * B, B)]], buf, sem,
                             add=True)

    def s3_x(c, buf, sem):
        @pl.when(ok(c))
        def _():
            pltpu.make_async_copy(sh_zin.at[pl.ds(0, B)], buf, sem).wait()
            pltpu.async_copy(x_hbm.at[ridx.at[pl.ds(c * B, B)]], buf, sem,
                             add=True)

    def s4_out(c, buf, sem):
        @pl.when(ok(c))
        def _():
            pltpu.make_async_copy(x_hbm.at[pl.ds(0, B)], buf, sem).wait()
            pltpu.async_copy(buf, o_hbm.at[pl.ds((s0 + c) * B, B)], sem)

    def s5_drain(c, buf, sem):
        @pl.when(ok(c))
        def _():
            pltpu.make_async_copy(buf, o_hbm.at[pl.ds(0, B)], sem).wait()

    def block(t, _):
        for off in range(NBUF):
            p = t * NBUF + off
            s1_zin(p, acc[off], sems[off])
            s5_drain(p - 5, acc[(off + 1) % NBUF], sems[(off + 1) % NBUF])
            s4_out(p - 3, acc[(off + 3) % NBUF], sems[(off + 3) % NBUF])
            s3_x(p - 2, acc[(off + 4) % NBUF], sems[(off + 4) % NBUF])
            s2_zout(p - 1, acc[(off + 5) % NBUF], sems[(off + 5) % NBUF])
        return 0

    lax.fori_loop(0, BLOCKS, block, 0)

    @pl.when(wid == NW - 1)
    def _tail():
        base = C * B
        pltpu.sync_copy(ind_hbm.at[pl.ds(base, TAIL)], tidx_i)
        pltpu.sync_copy(outd_hbm.at[pl.ds(base, TAIL)], tidx_o)
        pltpu.sync_copy(rid_hbm.at[pl.ds(base, TAIL)], tridx)
        tacc = acc[0].at[pl.ds(0, TAIL)]
        pltpu.sync_copy(sh_zin.at[tidx_i], tacc)
        pltpu.sync_copy(sh_zout.at[tidx_o], tacc, add=True)
        pltpu.sync_copy(x_hbm.at[tridx], tacc, add=True)
        pltpu.sync_copy(tacc, o_hbm.at[pl.ds(base, TAIL)])


def kernel(x, in_degree, out_degree, z_in, z_out):
    row_ids = jnp.arange(N, dtype=jnp.int32)
    return _sc_kernel(x, in_degree.astype(jnp.int32),
                      out_degree.astype(jnp.int32), row_ids, z_in, z_out)


# speedup vs baseline: 6.1633x; 1.0526x over previous
import functools

import jax
import jax.numpy as jnp
from jax import lax
from jax.experimental import pallas as pl
from jax.experimental.pallas import tpu as pltpu
from jax.experimental.pallas import tpu_sc as plsc

N = 100000
D = 128
B = 160
C = N // B
NC = 2
NS = 16
NW = NC * NS
BIG = C - (C // NW) * NW
CPW_BIG = C // NW + 1
CPW_SMALL = C // NW
NBUF = 5
BLOCKS = (CPW_BIG + 4 + NBUF - 1) // NBUF
V = 512
LANES = 16

_mesh = plsc.VectorSubcoreMesh(core_axis_name="c", subcore_axis_name="s")


@functools.partial(
    pl.kernel,
    out_type=jax.ShapeDtypeStruct((N, D), jnp.float32),
    mesh=_mesh,
    scratch_types=[
        pltpu.VMEM((CPW_BIG * B,), jnp.int32),
        pltpu.VMEM((CPW_BIG * B,), jnp.int32),
        pltpu.VMEM((CPW_BIG * B,), jnp.int32),
        [pltpu.VMEM((B, D), jnp.float32) for _ in range(NBUF)],
        [pltpu.SemaphoreType.DMA for _ in range(NBUF)],
        pltpu.VMEM_SHARED((V, D), jnp.float32),
        pltpu.VMEM_SHARED((V, D), jnp.float32),
    ],
)
def _sc_kernel(x_hbm, ind_hbm, outd_hbm, zin_hbm, zout_hbm, o_hbm,
               idx_i, idx_o, ridx, acc, sems, sh_zin, sh_zout):
    cid = lax.axis_index("c")
    sid = lax.axis_index("s")
    wid = sid * NC + cid
    is_big = wid < BIG
    s0 = jnp.where(is_big, wid * CPW_BIG, wid * CPW_SMALL + BIG)
    n = jnp.where(is_big, CPW_BIG, CPW_SMALL)

    @pl.when(is_big)
    def _idx_big():
        pltpu.async_copy(ind_hbm.at[pl.ds(s0 * B, CPW_BIG * B)], idx_i,
                         sems[0])
        pltpu.async_copy(outd_hbm.at[pl.ds(s0 * B, CPW_BIG * B)], idx_o,
                         sems[1])

    @pl.when(jnp.logical_not(is_big))
    def _idx_small():
        pltpu.async_copy(ind_hbm.at[pl.ds(s0 * B, CPW_SMALL * B)],
                         idx_i.at[pl.ds(0, CPW_SMALL * B)], sems[0])
        pltpu.async_copy(outd_hbm.at[pl.ds(s0 * B, CPW_SMALL * B)],
                         idx_o.at[pl.ds(0, CPW_SMALL * B)], sems[1])

    TS = V // NS
    pltpu.async_copy(zin_hbm.at[pl.ds(sid * TS, TS)],
                     sh_zin.at[pl.ds(sid * TS, TS)], sems[2])
    pltpu.async_copy(zout_hbm.at[pl.ds(sid * TS, TS)],
                     sh_zout.at[pl.ds(sid * TS, TS)], sems[3])

    iota16 = lax.broadcasted_iota(jnp.int32, (LANES,), 0)
    row0 = s0 * B + iota16

    def rid_body(g, _):
        ridx[pl.ds(g * LANES, LANES)] = row0 + g * LANES
        return 0

    lax.fori_loop(0, n * (B // LANES), rid_body, 0)

    @pl.when(is_big)
    def _w_big():
        pltpu.make_async_copy(ind_hbm.at[pl.ds(0, CPW_BIG * B)], idx_i,
                              sems[0]).wait()
        pltpu.make_async_copy(ind_hbm.at[pl.ds(0, CPW_BIG * B)], idx_o,
                              sems[1]).wait()

    @pl.when(jnp.logical_not(is_big))
    def _w_small():
        pltpu.make_async_copy(ind_hbm.at[pl.ds(0, CPW_SMALL * B)],
                              idx_i.at[pl.ds(0, CPW_SMALL * B)],
                              sems[0]).wait()
        pltpu.make_async_copy(ind_hbm.at[pl.ds(0, CPW_SMALL * B)],
                              idx_o.at[pl.ds(0, CPW_SMALL * B)],
                              sems[1]).wait()

    pltpu.make_async_copy(zin_hbm.at[pl.ds(0, TS)],
                          sh_zin.at[pl.ds(0, TS)], sems[2]).wait()
    pltpu.make_async_copy(zout_hbm.at[pl.ds(0, TS)],
                          sh_zout.at[pl.ds(0, TS)], sems[3]).wait()
    plsc.subcore_barrier()

    def ok(c):
        return jnp.logical_and(c >= 0, c < n)

    def s1_zin(c, buf, sem):
        @pl.when(ok(c))
        def _():
            pltpu.async_copy(sh_zin.at[idx_i.at[pl.ds(c * B, B)]], buf, sem)

    def s2_zout(c, buf, sem):
        @pl.when(ok(c))
        def _():
            pltpu.make_async_copy(sh_zin.at[pl.ds(0, B)], buf, sem).wait()
            pltpu.async_copy(sh_zout.at[idx_o.at[pl.ds(c * B, B)]], buf, sem,
                             add=True)

    def s3_x(c, buf, sem):
        @pl.when(ok(c))
        def _():
            pltpu.make_async_copy(sh_zin.at[pl.ds(0, B)], buf, sem).wait()
            pltpu.async_copy(x_hbm.at[ridx.at[pl.ds(c * B, B)]], buf, sem,
                             add=True)

    def s4_out(c, buf, sem):
        @pl.when(ok(c))
        def _():
            pltpu.make_async_copy(x_hbm.at[pl.ds(0, B)], buf, sem).wait()
            pltpu.async_copy(buf, o_hbm.at[pl.ds((s0 + c) * B, B)], sem)

    def s5_drain(c, buf, sem):
        @pl.when(ok(c))
        def _():
            pltpu.make_async_copy(buf, o_hbm.at[pl.ds(0, B)], sem).wait()

    def block(t, _):
        for off in range(NBUF):
            p = t * NBUF + off
            s5_drain(p - 5, acc[off], sems[off])
            s1_zin(p, acc[off], sems[off])
            s4_out(p - 3, acc[(off + 2) % NBUF], sems[(off + 2) % NBUF])
            s3_x(p - 2, acc[(off + 3) % NBUF], sems[(off + 3) % NBUF])
            s2_zout(p - 1, acc[(off + 4) % NBUF], sems[(off + 4) % NBUF])
        return 0

    lax.fori_loop(0, BLOCKS, block, 0)


def kernel(x, in_degree, out_degree, z_in, z_out):
    return _sc_kernel(x, in_degree.astype(jnp.int32),
                      out_degree.astype(jnp.int32), z_in, z_out)
